# initial kernel scaffold (unmeasured)
import jax
import jax.numpy as jnp
from jax import lax
from jax.experimental import pallas as pl
from jax.experimental.pallas import tpu as pltpu

N_DEV = 8
B = 1024
N_OUT = 4096


def kernel(x, w_mat):
    m_glob, k_sh = x.shape
    k_glob, n = w_mat.shape
    assert k_sh == B and m_glob == N_DEV * B and n == N_OUT

    def body(x_hbm, w_hbm, out_ref, recv_hbm, xk_vmem, w_vmem,
             send_sems, recv_sems, xk_copy_sems, w_copy_sems):
        me = lax.axis_index("i")

        barrier_sem = pltpu.get_barrier_semaphore()
        for k in range(1, N_DEV):
            pl.semaphore_signal(
                barrier_sem, inc=1,
                device_id=((me + k) % N_DEV,),
                device_id_type=pl.DeviceIdType.MESH,
            )
        pl.semaphore_wait(barrier_sem, N_DEV - 1)

        rdmas = []
        for k in range(1, N_DEV):
            dst = (me - k) % N_DEV
            rdma = pltpu.make_async_remote_copy(
                src_ref=x_hbm.at[pl.ds(dst * B, B), :],
                dst_ref=recv_hbm.at[k],
                send_sem=send_sems.at[k],
                recv_sem=recv_sems.at[k],
                device_id=(dst,),
                device_id_type=pl.DeviceIdType.MESH,
            )
            rdma.start()
            rdmas.append(rdma)

        for t in range(N_DEV):
            buf = t % 2
            s = (me + t) % N_DEV
            if t == 0:
                xk_src = x_hbm.at[pl.ds(me * B, B), :]
            else:
                rdmas[t - 1].wait_recv()
                xk_src = recv_hbm.at[t]
            xk_cp = pltpu.make_async_copy(
                xk_src, xk_vmem.at[buf], xk_copy_sems.at[buf])
            xk_cp.start()
            w_cp = pltpu.make_async_copy(
                w_hbm.at[pl.ds(s * B, B), :], w_vmem.at[buf],
                w_copy_sems.at[buf])
            w_cp.start()
            xk_cp.wait()
            w_cp.wait()
            part = lax.dot_general(
                xk_vmem[buf].astype(jnp.bfloat16),
                w_vmem[buf].astype(jnp.bfloat16),
                (((1,), (0,)), ((), ())),
                preferred_element_type=jnp.float32,
            )
            if t == 0:
                out_ref[...] = part
            else:
                out_ref[...] = out_ref[...] + part

        y = out_ref[...]
        c = 0.7978845608028654
        out_ref[...] = 0.5 * y * (1.0 + jnp.tanh(c * (y + 0.044715 * y * y * y)))

        for rdma in rdmas:
            rdma.wait_send()

    return pl.pallas_call(
        body,
        out_shape=jax.ShapeDtypeStruct((B, N_OUT), jnp.float32),
        in_specs=[
            pl.BlockSpec(memory_space=pltpu.ANY),
            pl.BlockSpec(memory_space=pltpu.ANY),
        ],
        out_specs=pl.BlockSpec(memory_space=pltpu.VMEM),
        scratch_shapes=[
            pltpu.ANY((N_DEV, B, B), jnp.float32),
            pltpu.VMEM((2, B, B), jnp.float32),
            pltpu.VMEM((2, B, N_OUT), jnp.float32),
            pltpu.SemaphoreType.DMA((N_DEV,)),
            pltpu.SemaphoreType.DMA((N_DEV,)),
            pltpu.SemaphoreType.DMA((2,)),
            pltpu.SemaphoreType.DMA((2,)),
        ],
        compiler_params=pltpu.CompilerParams(collective_id=0),
    )(x, w_mat)


# baseline (device time: 329470 ns/iter reference)
import jax
import jax.numpy as jnp
from jax import lax
from jax.experimental import pallas as pl
from jax.experimental.pallas import tpu as pltpu

N_DEV = 8
B = 1024
N_OUT = 4096


def kernel(x, w_mat):
    m_glob, k_sh = x.shape
    k_glob, n = w_mat.shape
    assert k_sh == B and m_glob == N_DEV * B and n == N_OUT

    def body(x_hbm, w_hbm, out_ref, recv_hbm, xk_vmem, w_vmem,
             send_sems, recv_sems, xk_copy_sems, w_copy_sems):
        me = lax.axis_index("i")

        barrier_sem = pltpu.get_barrier_semaphore()
        for k in range(1, N_DEV):
            pl.semaphore_signal(
                barrier_sem, inc=1,
                device_id=((me + k) % N_DEV,),
                device_id_type=pl.DeviceIdType.MESH,
            )
        pl.semaphore_wait(barrier_sem, N_DEV - 1)

        rdmas = []
        for k in range(1, N_DEV):
            dst = (me - k) % N_DEV
            rdma = pltpu.make_async_remote_copy(
                src_ref=x_hbm.at[pl.ds(dst * B, B), :],
                dst_ref=recv_hbm.at[k],
                send_sem=send_sems.at[k],
                recv_sem=recv_sems.at[k],
                device_id=(dst,),
                device_id_type=pl.DeviceIdType.MESH,
            )
            rdma.start()
            rdmas.append(rdma)

        for t in range(N_DEV):
            buf = t % 2
            s = (me + t) % N_DEV
            if t == 0:
                xk_src = x_hbm.at[pl.ds(me * B, B), :]
            else:
                rdmas[t - 1].wait_recv()
                xk_src = recv_hbm.at[t]
            xk_cp = pltpu.make_async_copy(
                xk_src, xk_vmem.at[buf], xk_copy_sems.at[buf])
            xk_cp.start()
            w_cp = pltpu.make_async_copy(
                w_hbm.at[pl.ds(s * B, B), :], w_vmem.at[buf],
                w_copy_sems.at[buf])
            w_cp.start()
            xk_cp.wait()
            w_cp.wait()
            part = lax.dot_general(
                xk_vmem[buf].astype(jnp.bfloat16),
                w_vmem[buf].astype(jnp.bfloat16),
                (((1,), (0,)), ((), ())),
                preferred_element_type=jnp.float32,
            )
            if t == 0:
                out_ref[...] = part
            else:
                out_ref[...] = out_ref[...] + part

        y = out_ref[...]
        c = 0.7978845608028654
        out_ref[...] = 0.5 * y * (1.0 + jnp.tanh(c * (y + 0.044715 * y * y * y)))

        for rdma in rdmas:
            rdma.wait_send()

    out, _recv = pl.pallas_call(
        body,
        out_shape=[
            jax.ShapeDtypeStruct((B, N_OUT), jnp.float32),
            jax.ShapeDtypeStruct((N_DEV, B, B), jnp.float32),
        ],
        in_specs=[
            pl.BlockSpec(memory_space=pltpu.HBM),
            pl.BlockSpec(memory_space=pltpu.HBM),
        ],
        out_specs=[
            pl.BlockSpec(memory_space=pltpu.VMEM),
            pl.BlockSpec(memory_space=pltpu.HBM),
        ],
        scratch_shapes=[
            pltpu.VMEM((2, B, B), jnp.float32),
            pltpu.VMEM((2, B, N_OUT), jnp.float32),
            pltpu.SemaphoreType.DMA((N_DEV,)),
            pltpu.SemaphoreType.DMA((N_DEV,)),
            pltpu.SemaphoreType.DMA((2,)),
            pltpu.SemaphoreType.DMA((2,)),
        ],
        compiler_params=pltpu.CompilerParams(
            collective_id=0,
            vmem_limit_bytes=64 * 1024 * 1024,
        ),
    )(x, w_mat)
    return out


# device time: 200810 ns/iter; 1.6407x vs baseline; 1.6407x over previous
import jax
import jax.numpy as jnp
from jax import lax
from jax.experimental import pallas as pl
from jax.experimental.pallas import tpu as pltpu

N_DEV = 8
B = 1024
N_OUT = 4096
HN = N_OUT // 2


def kernel(x, w_mat):
    m_glob, k_sh = x.shape
    k_glob, n = w_mat.shape
    assert k_sh == B and m_glob == N_DEV * B and n == N_OUT

    x16 = x.astype(jnp.bfloat16)

    def body(x16_hbm, w_hbm, out_ref, recv_vmem, w_vmem,
             send_sems, recv_sems, w_sems, loc_sem):
        me = lax.axis_index("i")

        loc_cp = pltpu.make_async_copy(
            x16_hbm.at[pl.ds(me * B, B), :], recv_vmem.at[0], loc_sem)
        loc_cp.start()

        barrier_sem = pltpu.get_barrier_semaphore()
        for k in range(1, N_DEV):
            pl.semaphore_signal(
                barrier_sem, inc=1,
                device_id=((me + k) % N_DEV,),
                device_id_type=pl.DeviceIdType.MESH,
            )
        pl.semaphore_wait(barrier_sem, N_DEV - 1)

        rdmas = []
        for k in range(1, N_DEV):
            dst = (me - k) % N_DEV
            rdma = pltpu.make_async_remote_copy(
                src_ref=x16_hbm.at[pl.ds(dst * B, B), :],
                dst_ref=recv_vmem.at[k],
                send_sem=send_sems.at[k],
                recv_sem=recv_sems.at[k],
                device_id=(dst,),
                device_id_type=pl.DeviceIdType.MESH,
            )
            rdma.start()
            rdmas.append(rdma)

        def w_copy(u):
            t, h = divmod(u, 2)
            s = (me + t) % N_DEV
            return pltpu.make_async_copy(
                w_hbm.at[pl.ds(s * B, B), pl.ds(h * HN, HN)],
                w_vmem.at[u % 2],
                w_sems.at[u % 2],
            )

        w_copy(0).start()
        for u in range(2 * N_DEV):
            t, h = divmod(u, 2)
            if u + 1 < 2 * N_DEV:
                w_copy(u + 1).start()
            if h == 0:
                if t == 0:
                    loc_cp.wait()
                else:
                    rdmas[t - 1].wait_recv()
            w_copy(u).wait()
            part = lax.dot_general(
                recv_vmem[t],
                w_vmem[u % 2].astype(jnp.bfloat16),
                (((1,), (0,)), ((), ())),
                preferred_element_type=jnp.float32,
            )
            if t == 0:
                out_ref[:, h * HN:(h + 1) * HN] = part
            else:
                out_ref[:, h * HN:(h + 1) * HN] += part

        c = 0.7978845608028654
        for g in range(4):
            sl = pl.ds(g * (N_OUT // 4), N_OUT // 4)
            y = out_ref[:, sl]
            out_ref[:, sl] = 0.5 * y * (
                1.0 + jnp.tanh(c * (y + 0.044715 * y * y * y)))

        for rdma in rdmas:
            rdma.wait_send()

    return pl.pallas_call(
        body,
        out_shape=jax.ShapeDtypeStruct((B, N_OUT), jnp.float32),
        in_specs=[
            pl.BlockSpec(memory_space=pltpu.HBM),
            pl.BlockSpec(memory_space=pltpu.HBM),
        ],
        out_specs=pl.BlockSpec(memory_space=pltpu.VMEM),
        scratch_shapes=[
            pltpu.VMEM((N_DEV, B, B), jnp.bfloat16),
            pltpu.VMEM((2, B, HN), jnp.float32),
            pltpu.SemaphoreType.DMA((N_DEV,)),
            pltpu.SemaphoreType.DMA((N_DEV,)),
            pltpu.SemaphoreType.DMA((2,)),
            pltpu.SemaphoreType.DMA,
        ],
        compiler_params=pltpu.CompilerParams(
            collective_id=0,
            vmem_limit_bytes=64 * 1024 * 1024,
        ),
    )(x16, w_mat)


# device time: 183233 ns/iter; 1.7981x vs baseline; 1.0959x over previous
import jax
import jax.numpy as jnp
from jax import lax
from jax.experimental import pallas as pl
from jax.experimental.pallas import tpu as pltpu

N_DEV = 8
B = 1024
N_OUT = 4096
HN = N_OUT // 2


def kernel(x, w_mat):
    m_glob, k_sh = x.shape
    k_glob, n = w_mat.shape
    assert k_sh == B and m_glob == N_DEV * B and n == N_OUT

    def body(x_hbm, w_hbm, out_ref, x16_hbm, recv_vmem, w_vmem, c16_vmem,
             send_sems, recv_sems, w_sems, co_sems, loc_sem):
        me = lax.axis_index("i")

        barrier_sem = pltpu.get_barrier_semaphore()
        for k in range(1, N_DEV):
            pl.semaphore_signal(
                barrier_sem, inc=1,
                device_id=((me + k) % N_DEV,),
                device_id_type=pl.DeviceIdType.MESH,
            )

        def cast_in(k):
            dst = (me - k) % N_DEV
            return pltpu.make_async_copy(
                x_hbm.at[pl.ds(dst * B, B), :],
                w_vmem.at[k % 2, :, pl.ds(0, B)],
                w_sems.at[k % 2],
            )

        loc_cp = pltpu.make_async_copy(
            x_hbm.at[pl.ds(me * B, B), :],
            w_vmem.at[0, :, pl.ds(0, B)], loc_sem)
        loc_cp.start()
        cast_in(1).start()
        loc_cp.wait()
        recv_vmem[0] = w_vmem[0, :, :B].astype(jnp.bfloat16)

        pl.semaphore_wait(barrier_sem, N_DEV - 1)

        rdmas = []
        for k in range(1, N_DEV):
            buf = k % 2
            dst = (me - k) % N_DEV
            if k + 1 < N_DEV:
                cast_in(k + 1).start()
            cast_in(k).wait()
            c16_vmem[buf] = w_vmem[buf, :, :B].astype(jnp.bfloat16)
            co = pltpu.make_async_copy(
                c16_vmem.at[buf], x16_hbm.at[pl.ds(dst * B, B), :],
                co_sems.at[buf])
            co.start()
            co.wait()
            rdma = pltpu.make_async_remote_copy(
                src_ref=x16_hbm.at[pl.ds(dst * B, B), :],
                dst_ref=recv_vmem.at[k],
                send_sem=send_sems.at[k],
                recv_sem=recv_sems.at[k],
                device_id=(dst,),
                device_id_type=pl.DeviceIdType.MESH,
            )
            rdma.start()
            rdmas.append(rdma)

        def w_copy(u):
            t, h = divmod(u, 2)
            s = (me + t) % N_DEV
            return pltpu.make_async_copy(
                w_hbm.at[pl.ds(s * B, B), pl.ds(h * HN, HN)],
                w_vmem.at[u % 2],
                w_sems.at[u % 2],
            )

        w_copy(0).start()
        for u in range(2 * N_DEV):
            t, h = divmod(u, 2)
            if u + 1 < 2 * N_DEV:
                w_copy(u + 1).start()
            if h == 0 and t > 0:
                rdmas[t - 1].wait_recv()
            w_copy(u).wait()
            part = lax.dot_general(
                recv_vmem[t],
                w_vmem[u % 2].astype(jnp.bfloat16),
                (((1,), (0,)), ((), ())),
                preferred_element_type=jnp.float32,
            )
            if t == 0:
                out_ref[:, h * HN:(h + 1) * HN] = part
            else:
                out_ref[:, h * HN:(h + 1) * HN] += part

        c = 0.7978845608028654
        for g in range(4):
            sl = pl.ds(g * (N_OUT // 4), N_OUT // 4)
            y = out_ref[:, sl]
            out_ref[:, sl] = 0.5 * y * (
                1.0 + jnp.tanh(c * (y + 0.044715 * y * y * y)))

        for rdma in rdmas:
            rdma.wait_send()

    out, _x16 = pl.pallas_call(
        body,
        out_shape=[
            jax.ShapeDtypeStruct((B, N_OUT), jnp.float32),
            jax.ShapeDtypeStruct((N_DEV * B, B), jnp.bfloat16),
        ],
        in_specs=[
            pl.BlockSpec(memory_space=pltpu.HBM),
            pl.BlockSpec(memory_space=pltpu.HBM),
        ],
        out_specs=[
            pl.BlockSpec(memory_space=pltpu.VMEM),
            pl.BlockSpec(memory_space=pltpu.HBM),
        ],
        scratch_shapes=[
            pltpu.VMEM((N_DEV, B, B), jnp.bfloat16),
            pltpu.VMEM((2, B, HN), jnp.float32),
            pltpu.VMEM((2, B, B), jnp.bfloat16),
            pltpu.SemaphoreType.DMA((N_DEV,)),
            pltpu.SemaphoreType.DMA((N_DEV,)),
            pltpu.SemaphoreType.DMA((2,)),
            pltpu.SemaphoreType.DMA((2,)),
            pltpu.SemaphoreType.DMA,
        ],
        compiler_params=pltpu.CompilerParams(
            collective_id=0,
            vmem_limit_bytes=64 * 1024 * 1024,
        ),
    )(x, w_mat)
    return out


# device time: 179933 ns/iter; 1.8311x vs baseline; 1.0183x over previous
import jax
import jax.numpy as jnp
from jax import lax
from jax.experimental import pallas as pl
from jax.experimental.pallas import tpu as pltpu

N_DEV = 8
B = 1024
N_OUT = 4096
HN = N_OUT // 2


def kernel(x, w_mat):
    m_glob, k_sh = x.shape
    k_glob, n = w_mat.shape
    assert k_sh == B and m_glob == N_DEV * B and n == N_OUT

    def body(x_hbm, w_hbm, out_ref, x16_hbm, recv_vmem, w_vmem, c16_vmem,
             send_sems, recv_sems, w_sems, co_sems, loc_sem):
        me = lax.axis_index("i")

        barrier_sem = pltpu.get_barrier_semaphore()
        for k in range(1, N_DEV):
            pl.semaphore_signal(
                barrier_sem, inc=1,
                device_id=((me + k) % N_DEV,),
                device_id_type=pl.DeviceIdType.MESH,
            )

        def cast_in(k):
            dst = (me - k) % N_DEV
            return pltpu.make_async_copy(
                x_hbm.at[pl.ds(dst * B, B), :],
                w_vmem.at[k % 2, :, pl.ds(0, B)],
                w_sems.at[k % 2],
            )

        loc_cp = pltpu.make_async_copy(
            x_hbm.at[pl.ds(me * B, B), :],
            w_vmem.at[0, :, pl.ds(0, B)], loc_sem)
        loc_cp.start()
        cast_in(1).start()
        loc_cp.wait()
        recv_vmem[0] = w_vmem[0, :, :B].astype(jnp.bfloat16)

        pl.semaphore_wait(barrier_sem, N_DEV - 1)

        rdmas = []
        for k in range(1, N_DEV):
            buf = k % 2
            dst = (me - k) % N_DEV
            if k + 1 < N_DEV:
                cast_in(k + 1).start()
            cast_in(k).wait()
            c16_vmem[buf] = w_vmem[buf, :, :B].astype(jnp.bfloat16)
            co = pltpu.make_async_copy(
                c16_vmem.at[buf], x16_hbm.at[pl.ds(dst * B, B), :],
                co_sems.at[buf])
            co.start()
            co.wait()
            rdma = pltpu.make_async_remote_copy(
                src_ref=x16_hbm.at[pl.ds(dst * B, B), :],
                dst_ref=recv_vmem.at[k],
                send_sem=send_sems.at[k],
                recv_sem=recv_sems.at[k],
                device_id=(dst,),
                device_id_type=pl.DeviceIdType.MESH,
            )
            rdma.start()
            rdmas.append(rdma)

        def w_copy(u):
            t, h = divmod(u, 2)
            s = (me + t) % N_DEV
            return pltpu.make_async_copy(
                w_hbm.at[pl.ds(s * B, B), pl.ds(h * HN, HN)],
                w_vmem.at[u % 2],
                w_sems.at[u % 2],
            )

        c = 0.7978845608028654

        def gelu(y):
            return 0.5 * y * (1.0 + jnp.tanh(c * (y + 0.044715 * y * y * y)))

        w_copy(0).start()
        for u in range(2 * N_DEV):
            t, h = divmod(u, 2)
            if u + 1 < 2 * N_DEV:
                w_copy(u + 1).start()
            if h == 0 and t > 0:
                rdmas[t - 1].wait_recv()
            w_copy(u).wait()
            part = lax.dot_general(
                recv_vmem[t],
                w_vmem[u % 2].astype(jnp.bfloat16),
                (((1,), (0,)), ((), ())),
                preferred_element_type=jnp.float32,
            )
            if t == 0:
                out_ref[:, h * HN:(h + 1) * HN] = part
            elif t < N_DEV - 1:
                out_ref[:, h * HN:(h + 1) * HN] += part
            else:
                for g in range(2):
                    lo = g * (HN // 2)
                    sl = slice(h * HN + lo, h * HN + lo + HN // 2)
                    out_ref[:, sl] = gelu(
                        out_ref[:, sl] + part[:, lo:lo + HN // 2])

        for rdma in rdmas:
            rdma.wait_send()

    out, _x16 = pl.pallas_call(
        body,
        out_shape=[
            jax.ShapeDtypeStruct((B, N_OUT), jnp.float32),
            jax.ShapeDtypeStruct((N_DEV * B, B), jnp.bfloat16),
        ],
        in_specs=[
            pl.BlockSpec(memory_space=pltpu.HBM),
            pl.BlockSpec(memory_space=pltpu.HBM),
        ],
        out_specs=[
            pl.BlockSpec(memory_space=pltpu.VMEM),
            pl.BlockSpec(memory_space=pltpu.HBM),
        ],
        scratch_shapes=[
            pltpu.VMEM((N_DEV, B, B), jnp.bfloat16),
            pltpu.VMEM((2, B, HN), jnp.float32),
            pltpu.VMEM((2, B, B), jnp.bfloat16),
            pltpu.SemaphoreType.DMA((N_DEV,)),
            pltpu.SemaphoreType.DMA((N_DEV,)),
            pltpu.SemaphoreType.DMA((2,)),
            pltpu.SemaphoreType.DMA((2,)),
            pltpu.SemaphoreType.DMA,
        ],
        compiler_params=pltpu.CompilerParams(
            collective_id=0,
            vmem_limit_bytes=64 * 1024 * 1024,
        ),
    )(x, w_mat)
    return out
